# baseline (speedup 1.0000x reference)
import functools

import jax
import jax.numpy as jnp
from jax import lax
from jax.experimental import pallas as pl
from jax.experimental.pallas import tpu as pltpu
from jax.experimental.pallas import tpu_sc as plsc

VOCAB = 1000000
D = 64
NC, NS = 2, 16
NW = NC * NS
N_J = 200
N_C = 32
ROW = 128

_mesh = plsc.VectorSubcoreMesh(core_axis_name="c", subcore_axis_name="s")


@functools.partial(
    pl.kernel,
    out_type=jax.ShapeDtypeStruct((N_J, 8, N_C, 8, ROW), jnp.float32),
    mesh=_mesh,
    compiler_params=pltpu.CompilerParams(
        use_tc_tiling_on_sc=False, needs_layout_passes=False),
    scratch_types=[
        pltpu.VMEM((25, 8, ROW), jnp.int32),
        pltpu.VMEM((ROW, D), jnp.float32),
        pltpu.VMEM((ROW, D), jnp.float32),
        pltpu.VMEM((8, 8, ROW), jnp.float32),
        pltpu.VMEM((8, 8, ROW), jnp.float32),
        pltpu.SemaphoreType.DMA,
        pltpu.SemaphoreType.DMA,
        pltpu.SemaphoreType.DMA,
        pltpu.SemaphoreType.DMA,
    ],
)
def _embed_gather(x4_hbm, tbl_hbm, out_hbm, idx_v, g0, g1, t0, t1,
                  semg0, semg1, sems0, sems1):
    w = lax.axis_index("s") * NC + lax.axis_index("c")

    for jh in range(25):
        pltpu.sync_copy(x4_hbm.at[jh, w], idx_v.at[jh])

    gbufs = (g0, g1)
    tbufs = (t0, t1)
    semgs = (semg0, semg1)
    semss = (sems0, sems1)

    iota16 = lax.iota(jnp.int32, 16)
    dhvecs = [(iota16 + 16 * m) // 8 for m in range(4)]
    dlvecs = [(iota16 + 16 * m) % 8 for m in range(4)]

    def fire_gather(j, p):
        pltpu.async_copy(
            tbl_hbm.at[idx_v.at[j // 8, j % 8]],
            gbufs[p], semgs[p])

    def wait_gather(j, p):
        pltpu.make_async_copy(
            tbl_hbm.at[idx_v.at[j // 8, j % 8]],
            gbufs[p], semgs[p]).wait()

    def transpose(p):
        g, t = gbufs[p], tbufs[p]
        zeros = jnp.zeros((16,), jnp.int32)

        def body(k, carry):
            r0 = k * 8
            tv = zeros + r0
            for s in range(8):
                for m in range(4):
                    v = g[r0 + s, pl.ds(16 * m, 16)]
                    plsc.store_scatter(t, [dhvecs[m], dlvecs[m], tv + s], v)
            return carry

        lax.fori_loop(0, ROW // 8, body, 0)

    def fire_store(j, p):
        pltpu.async_copy(tbufs[p], out_hbm.at[j, :, w], semss[p])

    def wait_store(j, p):
        pltpu.make_async_copy(tbufs[p], out_hbm.at[j, :, w], semss[p]).wait()

    def do_chunk(j, p):
        wait_gather(j, p)

        @pl.when(j < N_J - 1)
        def _():
            fire_gather(j + 1, 1 - p)

        @pl.when(j >= 2)
        def _():
            wait_store(j - 2, p)

        transpose(p)
        fire_store(j, p)

    fire_gather(0, 0)

    def step(k, carry):
        do_chunk(2 * k, 0)
        do_chunk(2 * k + 1, 1)
        return carry

    lax.fori_loop(0, N_J // 2, step, 0)
    wait_store(N_J - 2, 0)
    wait_store(N_J - 1, 1)


def kernel(x, table):
    x4 = x.T.reshape(25, 8, N_C, ROW).transpose(0, 2, 1, 3)
    tbl = table.reshape(VOCAB // 2, 2 * D).reshape(VOCAB, D)
    out5 = _embed_gather(x4, tbl)
    return out5.transpose(2, 4, 0, 1, 3).reshape(4096, N_J, D)
